# baseline (speedup 1.0000x reference)
import functools

import jax
import jax.numpy as jnp
from jax import lax
from jax.experimental import pallas as pl
from jax.experimental.pallas import tpu as pltpu
from jax.experimental.pallas import tpu_sc as plsc

_CHUNK = 80
_BATCH = 50
_LANES = 16


def _prep_body(a_ref, x_ref, w_ref, o_ref):
    a = jax.nn.sigmoid(a_ref[0])
    f = pl.program_id(0)
    scale = jnp.where(f == 0, a, 1.0 - a)
    o_ref[...] = scale * jnp.dot(x_ref[...], w_ref[0].T,
                                 preferred_element_type=jnp.float32)


def _finish_body(p_ref, b_ref, o_ref):
    s = p_ref[0] + p_ref[1] + b_ref[...]
    o_ref[...] = jnp.maximum(s, 0.0)


def _make_sc_spmm(N, D, E2):
    NC, NS = 2, 16
    per_tec = E2 // (NC * NS)
    assert per_tec % _CHUNK == 0
    n_chunks = per_tec // _CHUNK
    assert n_chunks % _BATCH == 0
    rows_per_tec = N // NS
    n_zero_chunks = -(-N // _CHUNK)
    zero_rounds = -(-n_zero_chunks // NS)
    groups = _CHUNK // _LANES

    mesh = plsc.VectorSubcoreMesh(core_axis_name="c", subcore_axis_name="s")

    @functools.partial(
        pl.kernel,
        out_type=jax.ShapeDtypeStruct((NC, N, D), jnp.float32),
        mesh=mesh,
        compiler_params=pltpu.CompilerParams(use_tc_tiling_on_sc=False),
        scratch_types=[
            pltpu.VMEM((_BATCH, _CHUNK), jnp.int32),
            pltpu.VMEM((_BATCH, _CHUNK), jnp.int32),
            pltpu.VMEM((_BATCH, _CHUNK), jnp.float32),
            pltpu.VMEM((_CHUNK, D // 2), jnp.int32),
            pltpu.VMEM((_CHUNK, D // 2), jnp.int32),
            pltpu.VMEM((_CHUNK, D), jnp.float32),
            pltpu.VMEM_SHARED((N, D), jnp.float32),
            pltpu.SemaphoreType.DMA,
            pltpu.SemaphoreType.DMA,
        ],
    )
    def sc_spmm(z_hbm, rows_hbm, cols_hbm, vals_hbm, out_hbm,
                col_v, row_v, val_v, gbuf0, gbuf1, sbuf, acc,
                gsem0, gsem1):
        c = lax.axis_index("c")
        s = lax.axis_index("s")
        wid = c * NS + s
        gbufs = (gbuf0, gbuf1)
        gsems = (gsem0, gsem1)

        def zero_buf(e, _):
            for j in range(D // _LANES):
                sbuf[e, pl.ds(j * _LANES, _LANES)] = jnp.zeros(
                    (_LANES,), jnp.float32)
            return _

        lax.fori_loop(0, _CHUNK, zero_buf, None)

        def zero_acc(k, _):
            m = s * zero_rounds + k

            @pl.when(m * _CHUNK < N)
            def _():
                pltpu.sync_copy(sbuf, acc.at[pl.ds(m * _CHUNK, _CHUNK)])

            return _

        lax.fori_loop(0, zero_rounds, zero_acc, None)
        plsc.subcore_barrier()

        lane_ids = [jnp.full((_LANES, 1), i, dtype=jnp.int32)
                    for i in range(_LANES)]
        _dnums = lax.GatherDimensionNumbers(
            offset_dims=(), collapsed_slice_dims=(0,), start_index_map=(0,))

        hi_mask = jnp.int32(-65536)

        def scale(gb, g):
            def scale_group(gg, _c):
                vv = val_v[g, pl.ds(gg * _LANES, _LANES)]
                for i in range(_LANES):
                    vb = lax.gather(
                        vv, lane_ids[i], _dnums, slice_sizes=(1,),
                        mode=lax.GatherScatterMode.PROMISE_IN_BOUNDS)
                    e = gg * _LANES + i
                    for t in range(D // 32):
                        w = gb[e, pl.ds(t * _LANES, _LANES)]
                        lo = lax.bitcast_convert_type(
                            lax.shift_left(w, 16), jnp.float32)
                        hi = lax.bitcast_convert_type(
                            lax.bitwise_and(w, hi_mask), jnp.float32)
                        sbuf[e, pl.ds(t * 32, _LANES)] = lo * vb
                        sbuf[e, pl.ds(t * 32 + _LANES, _LANES)] = hi * vb
                return _c

            lax.fori_loop(0, groups, scale_group, None)

        def batch_body(bt, _):
            pltpu.sync_copy(cols_hbm.at[wid, bt], col_v)
            pltpu.sync_copy(rows_hbm.at[wid, bt], row_v)
            pltpu.sync_copy(vals_hbm.at[wid, bt], val_v)
            pltpu.async_copy(z_hbm.at[col_v.at[0]], gbuf0, gsem0)

            def pair_body(p, _c):
                for b in range(2):
                    g = 2 * p + b
                    ob = 1 - b
                    gnext = jnp.minimum(g + 1, _BATCH - 1)
                    pltpu.async_copy(z_hbm.at[col_v.at[gnext]], gbufs[ob],
                                     gsems[ob])
                    pltpu.make_async_copy(z_hbm.at[col_v.at[g]], gbufs[b],
                                          gsems[b]).wait()
                    scale(gbufs[b], g)
                    pltpu.sync_copy(sbuf, acc.at[row_v.at[g]], add=True)
                return _c

            lax.fori_loop(0, _BATCH // 2, pair_body, None)
            pltpu.make_async_copy(z_hbm.at[col_v.at[0]], gbuf0, gsem0).wait()
            return _

        lax.fori_loop(0, n_chunks // _BATCH, batch_body, None)
        plsc.subcore_barrier()

        def write_out(k, _):
            m = s * zero_rounds + k

            @pl.when(m * _CHUNK < N)
            def _():
                pltpu.sync_copy(acc.at[pl.ds(m * _CHUNK, _CHUNK)],
                                out_hbm.at[c, pl.ds(m * _CHUNK, _CHUNK)])

            return _

        lax.fori_loop(0, zero_rounds, write_out, None)

    return sc_spmm


def kernel(x, lp_index, lp_values, hp_index, hp_values, W_LP, W_HP,
           alpha_raw, bias):
    N, D = x.shape
    E = lp_values.shape[0]
    BN = 2000
    NB = N // BN

    Ws = jnp.stack([W_LP, W_HP])

    Z = pl.pallas_call(
        _prep_body,
        grid=(2, NB),
        in_specs=[
            pl.BlockSpec(memory_space=pltpu.SMEM),
            pl.BlockSpec((BN, D), lambda f, b: (b, 0)),
            pl.BlockSpec((1, D, D), lambda f, b: (f, 0, 0)),
        ],
        out_specs=pl.BlockSpec((BN, D), lambda f, b: (f * NB + b, 0)),
        out_shape=jax.ShapeDtypeStruct((2 * N, D), jnp.float32),
    )(alpha_raw, x, Ws)

    Zb = Z.astype(jnp.bfloat16).reshape(2 * N, D // 32, 2, _LANES)
    Zp = lax.bitcast_convert_type(
        jnp.stack([Zb[:, :, 0, :], Zb[:, :, 1, :]], axis=-1), jnp.int32
    ).reshape(2 * N, D // 2)

    NW = 32
    quantum = NW * _CHUNK * _BATCH
    nbt = -(-(2 * E) // quantum)
    E2p = nbt * quantum
    pad = E2p - 2 * E
    shp = (NW, nbt, _BATCH, _CHUNK)
    ipad = jnp.zeros((pad,), jnp.int32)
    fpad = jnp.zeros((pad,), jnp.float32)
    rows = jnp.concatenate([lp_index[0], hp_index[0], ipad]).reshape(shp)
    cols = jnp.concatenate([lp_index[1], hp_index[1] + N, ipad]).reshape(shp)
    vals = jnp.concatenate([lp_values, hp_values, fpad]).reshape(shp)

    parts = _make_sc_spmm(N, D, E2p)(Zp, rows, cols, vals)

    out = pl.pallas_call(
        _finish_body,
        grid=(NB,),
        in_specs=[
            pl.BlockSpec((2, BN, D), lambda b: (0, b, 0)),
            pl.BlockSpec((1, D), lambda b: (0, 0)),
        ],
        out_specs=pl.BlockSpec((BN, D), lambda b: (b, 0)),
        out_shape=jax.ShapeDtypeStruct((N, D), jnp.float32),
    )(parts, bias.reshape(1, D))

    return out


# speedup vs baseline: 1.9453x; 1.9453x over previous
import functools

import jax
import jax.numpy as jnp
from jax import lax
from jax.experimental import pallas as pl
from jax.experimental.pallas import tpu as pltpu
from jax.experimental.pallas import tpu_sc as plsc

_CHUNK = 80
_BATCH = 50
_LANES = 16


def _prep_body(a_ref, x_ref, w_ref, o_ref):
    a = jax.nn.sigmoid(a_ref[0])
    f = pl.program_id(0)
    scale = jnp.where(f == 0, a, 1.0 - a)
    o_ref[...] = scale * jnp.dot(x_ref[...], w_ref[0].T,
                                 preferred_element_type=jnp.float32)


def _finish_body(p_ref, b_ref, o_ref):
    s = p_ref[0] + p_ref[1] + b_ref[...]
    o_ref[...] = jnp.maximum(s, 0.0)


def _make_sc_spmm(N, D, E2):
    NC, NS = 2, 16
    per_tec = E2 // (NC * NS)
    assert per_tec % _CHUNK == 0
    n_chunks = per_tec // _CHUNK
    assert n_chunks % _BATCH == 0
    rows_per_tec = N // NS
    n_zero_chunks = -(-N // _CHUNK)
    zero_rounds = -(-n_zero_chunks // NS)
    groups = _CHUNK // _LANES

    mesh = plsc.VectorSubcoreMesh(core_axis_name="c", subcore_axis_name="s")

    @functools.partial(
        pl.kernel,
        out_type=jax.ShapeDtypeStruct((NC, N, D), jnp.float32),
        mesh=mesh,
        scratch_types=[
            pltpu.VMEM((_BATCH, _CHUNK), jnp.int32),
            pltpu.VMEM((_BATCH, _CHUNK), jnp.int32),
            pltpu.VMEM((_BATCH, _CHUNK), jnp.float32),
            pltpu.VMEM((_CHUNK, D), jnp.float32),
            pltpu.VMEM((_CHUNK, D), jnp.float32),
            pltpu.VMEM_SHARED((N, D), jnp.float32),
            pltpu.SemaphoreType.DMA,
            pltpu.SemaphoreType.DMA,
        ],
    )
    def sc_spmm(z_hbm, rows_hbm, cols_hbm, vals_hbm, out_hbm,
                col_v, row_v, val_v, gbuf0, gbuf1, acc,
                gsem0, gsem1):
        c = lax.axis_index("c")
        s = lax.axis_index("s")
        wid = c * NS + s
        gbufs = (gbuf0, gbuf1)
        gsems = (gsem0, gsem1)

        def zero_buf(e, _):
            for j in range(D // _LANES):
                gbuf0[e, pl.ds(j * _LANES, _LANES)] = jnp.zeros(
                    (_LANES,), jnp.float32)
            return _

        lax.fori_loop(0, _CHUNK, zero_buf, None)

        def zero_acc(k, _):
            m = s * zero_rounds + k

            @pl.when(m * _CHUNK < N)
            def _():
                pltpu.sync_copy(gbuf0, acc.at[pl.ds(m * _CHUNK, _CHUNK)])

            return _

        lax.fori_loop(0, zero_rounds, zero_acc, None)
        plsc.subcore_barrier()

        lane_ids = [jnp.full((_LANES, 1), i, dtype=jnp.int32)
                    for i in range(_LANES)]
        _dnums = lax.GatherDimensionNumbers(
            offset_dims=(), collapsed_slice_dims=(0,), start_index_map=(0,))

        def scale(gb, g):
            def scale_group(gg, _c):
                vv = val_v[g, pl.ds(gg * _LANES, _LANES)]
                for i in range(_LANES):
                    vb = lax.gather(
                        vv, lane_ids[i], _dnums, slice_sizes=(1,),
                        mode=lax.GatherScatterMode.PROMISE_IN_BOUNDS)
                    e = gg * _LANES + i
                    for t in range(D // _LANES):
                        sl = pl.ds(t * _LANES, _LANES)
                        gb[e, sl] = gb[e, sl] * vb
                return _c

            lax.fori_loop(0, groups, scale_group, None)

        def batch_body(bt, _):
            pltpu.sync_copy(cols_hbm.at[wid, bt], col_v)
            pltpu.sync_copy(rows_hbm.at[wid, bt], row_v)
            pltpu.sync_copy(vals_hbm.at[wid, bt], val_v)
            pltpu.async_copy(z_hbm.at[col_v.at[0]], gbuf0, gsem0)

            def pair_body(p, _c):
                for b in range(2):
                    g = 2 * p + b
                    ob = 1 - b
                    gnext = jnp.minimum(g + 1, _BATCH - 1)
                    pltpu.async_copy(z_hbm.at[col_v.at[gnext]], gbufs[ob],
                                     gsems[ob])
                    pltpu.make_async_copy(z_hbm.at[col_v.at[g]], gbufs[b],
                                          gsems[b]).wait()
                    scale(gbufs[b], g)
                    pltpu.sync_copy(gbufs[b], acc.at[row_v.at[g]], add=True)
                return _c

            lax.fori_loop(0, _BATCH // 2, pair_body, None)
            pltpu.make_async_copy(z_hbm.at[col_v.at[0]], gbuf0, gsem0).wait()
            return _

        lax.fori_loop(0, n_chunks // _BATCH, batch_body, None)
        plsc.subcore_barrier()

        def write_out(k, _):
            m = s * zero_rounds + k

            @pl.when(m * _CHUNK < N)
            def _():
                pltpu.sync_copy(acc.at[pl.ds(m * _CHUNK, _CHUNK)],
                                out_hbm.at[c, pl.ds(m * _CHUNK, _CHUNK)])

            return _

        lax.fori_loop(0, zero_rounds, write_out, None)

    return sc_spmm


def kernel(x, lp_index, lp_values, hp_index, hp_values, W_LP, W_HP,
           alpha_raw, bias):
    N, D = x.shape
    E = lp_values.shape[0]
    BN = 2000
    NB = N // BN

    Ws = jnp.stack([W_LP, W_HP])

    Z = pl.pallas_call(
        _prep_body,
        grid=(2, NB),
        in_specs=[
            pl.BlockSpec(memory_space=pltpu.SMEM),
            pl.BlockSpec((BN, D), lambda f, b: (b, 0)),
            pl.BlockSpec((1, D, D), lambda f, b: (f, 0, 0)),
        ],
        out_specs=pl.BlockSpec((BN, D), lambda f, b: (f * NB + b, 0)),
        out_shape=jax.ShapeDtypeStruct((2 * N, D), jnp.float32),
    )(alpha_raw, x, Ws)

    Zp = Z
    NW = 32
    quantum = NW * _CHUNK * _BATCH
    nbt = -(-(2 * E) // quantum)
    E2p = nbt * quantum
    pad = E2p - 2 * E
    shp = (NW, nbt, _BATCH, _CHUNK)
    ipad = jnp.zeros((pad,), jnp.int32)
    fpad = jnp.zeros((pad,), jnp.float32)
    rows = jnp.concatenate([lp_index[0], hp_index[0], ipad]).reshape(shp)
    cols = jnp.concatenate([lp_index[1], hp_index[1] + N, ipad]).reshape(shp)
    vals = jnp.concatenate([lp_values, hp_values, fpad]).reshape(shp)

    parts = _make_sc_spmm(N, D, E2p)(Zp, rows, cols, vals)

    out = pl.pallas_call(
        _finish_body,
        grid=(NB,),
        in_specs=[
            pl.BlockSpec((2, BN, D), lambda b: (0, b, 0)),
            pl.BlockSpec((1, D), lambda b: (0, 0)),
        ],
        out_specs=pl.BlockSpec((BN, D), lambda b: (b, 0)),
        out_shape=jax.ShapeDtypeStruct((N, D), jnp.float32),
    )(parts, bias.reshape(1, D))

    return out
